# baseline (speedup 1.0000x reference)
import functools

import jax
import jax.numpy as jnp
from jax import lax
from jax.experimental import pallas as pl
from jax.experimental.pallas import tpu as pltpu
from jax.experimental.pallas import tpu_sc as plsc

SEQ = 200
BATCH = 4096
DIM = 64
VOCAB = 1000000
B = SEQ * BATCH
NC = 2
NS = 16
NW = NC * NS
B_PER_W = B // NW
SUB = 128
N_SUB = 8
CHUNK = SUB * N_SUB
N_CHUNKS = B_PER_W // CHUNK

_mesh = plsc.VectorSubcoreMesh(core_axis_name="c", subcore_axis_name="s")


@functools.partial(
    pl.kernel,
    mesh=_mesh,
    out_type=jax.ShapeDtypeStruct((B, DIM), jnp.float32),
    compiler_params=pltpu.CompilerParams(use_tc_tiling_on_sc=False),
    scratch_types=[
        pltpu.VMEM((N_SUB, SUB), jnp.int32),
        pltpu.VMEM((CHUNK, DIM), jnp.float32),
        pltpu.SemaphoreType.DMA,
    ],
)
def _gather_kernel(idx_hbm, table_hbm, out_hbm, idx_v, rows_v, sem):
    wid = lax.axis_index("s") * NC + lax.axis_index("c")
    base = wid * B_PER_W

    def body(g, carry):
        off = base + g * CHUNK
        idx_row = pl.multiple_of(off // SUB, 8)
        pltpu.sync_copy(idx_hbm.at[pl.ds(idx_row, N_SUB)], idx_v)
        copies = []
        for j in range(N_SUB):
            copies.append(
                pltpu.async_copy(
                    table_hbm.at[idx_v.at[j]],
                    rows_v.at[pl.ds(j * SUB, SUB)],
                    sem,
                )
            )
        for c in copies:
            c.wait()
        pltpu.sync_copy(rows_v, out_hbm.at[pl.ds(off, CHUNK)])
        return carry

    lax.fori_loop(0, N_CHUNKS, body, 0)


def kernel(source, table):
    idx = source.reshape(B // SUB, SUB)
    out = _gather_kernel(idx, table)
    return out.reshape(SEQ, BATCH, 1, DIM)


# speedup vs baseline: 1.0165x; 1.0165x over previous
import functools

import jax
import jax.numpy as jnp
from jax import lax
from jax.experimental import pallas as pl
from jax.experimental.pallas import tpu as pltpu
from jax.experimental.pallas import tpu_sc as plsc

SEQ = 200
BATCH = 4096
DIM = 64
VOCAB = 1000000
B = SEQ * BATCH
NC = 2
NS = 16
NW = NC * NS
B_PER_W = B // NW
SUB = 128
N_SUB = 4
HALF = SUB * N_SUB
N_HALVES = B_PER_W // HALF
IDX_ROWS = B_PER_W // SUB

_mesh = plsc.VectorSubcoreMesh(core_axis_name="c", subcore_axis_name="s")


@functools.partial(
    pl.kernel,
    mesh=_mesh,
    out_type=jax.ShapeDtypeStruct((B, DIM), jnp.float32),
    compiler_params=pltpu.CompilerParams(use_tc_tiling_on_sc=False),
    scratch_types=[
        pltpu.VMEM((IDX_ROWS, SUB), jnp.int32),
        pltpu.VMEM((HALF, DIM), jnp.float32),
        pltpu.VMEM((HALF, DIM), jnp.float32),
        pltpu.SemaphoreType.DMA,
        pltpu.SemaphoreType.DMA,
        pltpu.SemaphoreType.DMA,
        pltpu.SemaphoreType.DMA,
    ],
)
def _gather_kernel(idx_hbm, table_hbm, out_hbm, idx_v, rows0, rows1,
                   gs0, gs1, ws0, ws1):
    wid = lax.axis_index("s") * NC + lax.axis_index("c")
    base = wid * B_PER_W

    idx_base = pl.multiple_of(wid * IDX_ROWS, 8)
    pltpu.sync_copy(idx_hbm.at[pl.ds(idx_base, IDX_ROWS)], idx_v)

    def fire_gathers(h, buf, sem):
        for j in range(N_SUB):
            pltpu.async_copy(
                table_hbm.at[idx_v.at[h * N_SUB + j]],
                buf.at[pl.ds(j * SUB, SUB)],
                sem,
            )

    def wait_gathers(buf, sem):
        pltpu.make_async_copy(table_hbm.at[pl.ds(0, HALF)], buf, sem).wait()

    def fire_write(h, buf, sem):
        off = pl.multiple_of(base + h * HALF, 8)
        pltpu.async_copy(buf, out_hbm.at[pl.ds(off, HALF)], sem)

    def wait_write(buf, sem):
        pltpu.make_async_copy(buf, out_hbm.at[pl.ds(0, HALF)], sem).wait()

    fire_gathers(0, rows0, gs0)

    def body(k, carry):
        a = 2 * k
        wait_gathers(rows0, gs0)

        @pl.when(k > 0)
        def _():
            wait_write(rows1, ws1)

        fire_gathers(a + 1, rows1, gs1)
        fire_write(a, rows0, ws0)
        wait_gathers(rows1, gs1)
        wait_write(rows0, ws0)

        @pl.when(k < N_HALVES // 2 - 1)
        def _():
            fire_gathers(a + 2, rows0, gs0)

        fire_write(a + 1, rows1, ws1)
        return carry

    lax.fori_loop(0, N_HALVES // 2, body, 0)
    wait_write(rows1, ws1)


def kernel(source, table):
    idx = source.reshape(B // SUB, SUB)
    out = _gather_kernel(idx, table)
    return out.reshape(SEQ, BATCH, 1, DIM)


# speedup vs baseline: 1.0171x; 1.0005x over previous
import functools

import jax
import jax.numpy as jnp
from jax import lax
from jax.experimental import pallas as pl
from jax.experimental.pallas import tpu as pltpu
from jax.experimental.pallas import tpu_sc as plsc

SEQ = 200
BATCH = 4096
DIM = 64
VOCAB = 1000000
B = SEQ * BATCH
NC = 2
NS = 16
NW = NC * NS
B_PER_W = B // NW
SUB = 128
N_SUB = 2
CHUNK = SUB * N_SUB
NBUF = 4
H = B_PER_W // CHUNK
K = H // NBUF
IDX_ROWS = B_PER_W // SUB

_mesh = plsc.VectorSubcoreMesh(core_axis_name="c", subcore_axis_name="s")


@functools.partial(
    pl.kernel,
    mesh=_mesh,
    out_type=jax.ShapeDtypeStruct((B, DIM), jnp.float32),
    compiler_params=pltpu.CompilerParams(use_tc_tiling_on_sc=False),
    scratch_types=[
        pltpu.VMEM((IDX_ROWS, SUB), jnp.int32),
        [pltpu.VMEM((CHUNK, DIM), jnp.float32)] * NBUF,
        [pltpu.SemaphoreType.DMA] * NBUF,
        [pltpu.SemaphoreType.DMA] * NBUF,
    ],
)
def _gather_kernel(idx_hbm, table_hbm, out_hbm, idx_v, bufs, gsems, wsems):
    wid = lax.axis_index("s") * NC + lax.axis_index("c")
    base = wid * B_PER_W

    idx_base = pl.multiple_of(wid * IDX_ROWS, 8)
    pltpu.sync_copy(idx_hbm.at[pl.ds(idx_base, IDX_ROWS)], idx_v)

    def fire_gathers(h, b):
        for j in range(N_SUB):
            pltpu.async_copy(
                table_hbm.at[idx_v.at[h * N_SUB + j]],
                bufs[b].at[pl.ds(j * SUB, SUB)],
                gsems[b],
            )

    def wait_gathers(b):
        pltpu.make_async_copy(
            table_hbm.at[pl.ds(0, CHUNK)], bufs[b], gsems[b]).wait()

    def fire_write(h, b):
        off = pl.multiple_of(base + h * CHUNK, 8)
        pltpu.async_copy(bufs[b], out_hbm.at[pl.ds(off, CHUNK)], wsems[b])

    def wait_write(b):
        pltpu.make_async_copy(
            bufs[b], out_hbm.at[pl.ds(0, CHUNK)], wsems[b]).wait()

    for b in range(NBUF - 1):
        fire_gathers(b, b)

    def body(k, carry):
        for j in range(NBUF):
            h = k * NBUF + j
            nb = (j + NBUF - 1) % NBUF
            wait_gathers(j)
            if j == 0:
                @pl.when(k > 0)
                def _():
                    wait_write(nb)
            else:
                wait_write(nb)
            if j == 0:
                fire_gathers(h + NBUF - 1, nb)
            else:
                @pl.when(k < K - 1)
                def _():
                    fire_gathers(h + NBUF - 1, nb)
            fire_write(h, j)
        return carry

    lax.fori_loop(0, K, body, 0)
    wait_write(NBUF - 1)


def kernel(source, table):
    idx = source.reshape(B // SUB, SUB)
    out = _gather_kernel(idx, table)
    return out.reshape(SEQ, BATCH, 1, DIM)
